# baseline (speedup 1.0000x reference)
import jax
import jax.numpy as jnp
from jax.experimental import pallas as pl
from jax.experimental.pallas import tpu as pltpu


def _body(io_idx_ref, pair_idx_ref, row_ref, col_ref, io_ref, pair_ref, out_ref):
    g = pl.program_id(0)
    br = row_ref.shape[0]
    gd = col_ref.shape[0]
    d4 = row_ref.shape[1]
    row = row_ref[...]
    col = col_ref[...]
    io_v = io_ref[io_idx_ref[g], :]
    pr_v = pair_ref[pair_idx_ref[g], :]
    out_ref[0, :, :, 0:d4] = jnp.broadcast_to(row[:, None, :], (br, gd, d4))
    out_ref[0, :, :, d4:2 * d4] = jnp.broadcast_to(col[None, :, :], (br, gd, d4))
    out_ref[0, :, :, 2 * d4:3 * d4] = jnp.broadcast_to(
        io_v[None, None, :], (br, gd, d4))
    out_ref[0, :, :, 3 * d4:4 * d4] = jnp.broadcast_to(
        pr_v[None, None, :], (br, gd, d4))


def kernel(row_table, col_table, io_table, pair_table, num_grids, grid_dim):
    gd = row_table.shape[0]
    ng = pair_table.shape[0] - 1
    d4 = row_table.shape[-1]
    d = 4 * d4

    grid_indices = jnp.arange(ng, dtype=jnp.int32) + (
        jnp.asarray(num_grids, jnp.int32) - ng)
    io_idx = (grid_indices % 2).astype(jnp.int32)
    pair_idx = (grid_indices // 2).astype(jnp.int32)

    br = 32
    nr = gd // br

    grid_spec = pltpu.PrefetchScalarGridSpec(
        num_scalar_prefetch=2,
        grid=(ng, nr),
        in_specs=[
            pl.BlockSpec((br, d4), lambda g, r, *_: (r, 0)),
            pl.BlockSpec((gd, d4), lambda g, r, *_: (0, 0)),
            pl.BlockSpec(io_table.shape, lambda g, r, *_: (0, 0)),
            pl.BlockSpec(pair_table.shape, lambda g, r, *_: (0, 0)),
        ],
        out_specs=pl.BlockSpec((1, br, gd, d), lambda g, r, *_: (g, r, 0, 0)),
    )

    return pl.pallas_call(
        _body,
        grid_spec=grid_spec,
        out_shape=jax.ShapeDtypeStruct((ng, gd, gd, d), row_table.dtype),
    )(io_idx, pair_idx, row_table, col_table, io_table, pair_table)


# speedup vs baseline: 1.0165x; 1.0165x over previous
import jax
import jax.numpy as jnp
from jax.experimental import pallas as pl
from jax.experimental.pallas import tpu as pltpu


def _body(io_idx_ref, pair_idx_ref, row_ref, col_ref, io_ref, pair_ref, out_ref):
    g = pl.program_id(0)
    br = row_ref.shape[0]
    gd = col_ref.shape[0]
    d4 = row_ref.shape[1]
    row = row_ref[...]
    col = col_ref[...]
    io_v = io_ref[io_idx_ref[g], :]
    pr_v = pair_ref[pair_idx_ref[g], :]
    out_ref[0, :, :, 0:d4] = jnp.broadcast_to(row[:, None, :], (br, gd, d4))
    out_ref[0, :, :, d4:2 * d4] = jnp.broadcast_to(col[None, :, :], (br, gd, d4))
    out_ref[0, :, :, 2 * d4:3 * d4] = jnp.broadcast_to(
        io_v[None, None, :], (br, gd, d4))
    out_ref[0, :, :, 3 * d4:4 * d4] = jnp.broadcast_to(
        pr_v[None, None, :], (br, gd, d4))


def kernel(row_table, col_table, io_table, pair_table, num_grids, grid_dim):
    gd = row_table.shape[0]
    ng = pair_table.shape[0] - 1
    d4 = row_table.shape[-1]
    d = 4 * d4

    grid_indices = jnp.arange(ng, dtype=jnp.int32) + (
        jnp.asarray(num_grids, jnp.int32) - ng)
    io_idx = (grid_indices % 2).astype(jnp.int32)
    pair_idx = (grid_indices // 2).astype(jnp.int32)

    br = 16
    nr = gd // br

    grid_spec = pltpu.PrefetchScalarGridSpec(
        num_scalar_prefetch=2,
        grid=(ng, nr),
        in_specs=[
            pl.BlockSpec((br, d4), lambda g, r, *_: (r, 0)),
            pl.BlockSpec((gd, d4), lambda g, r, *_: (0, 0)),
            pl.BlockSpec(io_table.shape, lambda g, r, *_: (0, 0)),
            pl.BlockSpec(pair_table.shape, lambda g, r, *_: (0, 0)),
        ],
        out_specs=pl.BlockSpec((1, br, gd, d), lambda g, r, *_: (g, r, 0, 0)),
    )

    return pl.pallas_call(
        _body,
        grid_spec=grid_spec,
        out_shape=jax.ShapeDtypeStruct((ng, gd, gd, d), row_table.dtype),
    )(io_idx, pair_idx, row_table, col_table, io_table, pair_table)
